# Initial kernel scaffold; baseline (speedup 1.0000x reference)
import jax
import jax.numpy as jnp
from jax.experimental import pallas as pl


def kernel(x, edge_index, W, b):
    raise NotImplementedError("write your pallas kernel here")


# speedup vs baseline: 3.9609x; 3.9609x over previous
import functools

import jax
import jax.numpy as jnp
from jax import lax
from jax.experimental import pallas as pl
from jax.experimental.pallas import tpu as pltpu
from jax.experimental.pallas import tpu_sc as plsc

_N = 10000
_E = 320000
_D = 128
_NC = 2
_NS = 16
_NW = _NC * _NS
_EPW = _E // _NW
_K = 80
_NCHUNK = _EPW // _K
_NP = 10112
_RPT = _NP // _NS

_RCHUNK = tuple(min(i * _K, _RPT - _K) for i in range(8))


def _sc_scatter(row, col, x, iota_arr):
  mesh = plsc.VectorSubcoreMesh(core_axis_name="c", subcore_axis_name="s")

  @functools.partial(
      pl.kernel,
      mesh=mesh,
      out_type=[
          jax.ShapeDtypeStruct((_NC, _NP, _D), jnp.float32),
          jax.ShapeDtypeStruct((_NC, _NP, _D), jnp.float32),
      ],
      scratch_types=[
          pltpu.VMEM((_K,), jnp.int32),
          pltpu.VMEM((_K,), jnp.int32),
          pltpu.VMEM((_K,), jnp.int32),
          pltpu.VMEM((_K, _D), jnp.float32),
          pltpu.VMEM((_K, _D), jnp.float32),
          pltpu.VMEM((_K, _D), jnp.float32),
          pltpu.VMEM_SHARED((_NP, _D), jnp.float32),
          pltpu.SemaphoreType.DMA,
      ],
  )
  def k(row_h, col_h, x_h, iota_h, sum_h, deg_h,
        rowi, coli, iot, rows, zb, ob, acc, sem):
    cid = lax.axis_index("c")
    sid = lax.axis_index("s")
    wid = sid * _NC + cid

    zv = jnp.zeros((16,), jnp.float32)
    ov = jnp.ones((16,), jnp.float32)

    def initv(r, carry):
      for cc in range(_D // 16):
        zb[r, pl.ds(cc * 16, 16)] = zv
        ob[r, pl.ds(cc * 16, 16)] = ov
      return carry
    lax.fori_loop(0, _K, initv, 0)

    base_r = sid * _RPT
    ebase = wid * _EPW

    def zero_acc():
      for off_s in _RCHUNK:
        off = pl.multiple_of(base_r + off_s, 8)
        pltpu.sync_copy(iota_h.at[pl.ds(off, _K)], iot)
        pltpu.sync_copy(zb, acc.at[iot])

    def drain(out_h):
      for off_s in _RCHUNK:
        off = pl.multiple_of(base_r + off_s, 8)
        pltpu.sync_copy(iota_h.at[pl.ds(off, _K)], iot)
        pltpu.async_copy(acc.at[iot], rows, sem).wait()
        pltpu.sync_copy(rows, out_h.at[cid, pl.ds(off, _K)])

    zero_acc()
    plsc.subcore_barrier()

    def chunk_a(ci, carry):
      b = pl.multiple_of(ebase + ci * _K, 8)
      pltpu.sync_copy(row_h.at[pl.ds(b, _K)], rowi)
      pltpu.sync_copy(col_h.at[pl.ds(b, _K)], coli)
      pltpu.async_copy(x_h.at[coli], rows, sem).wait()
      pltpu.sync_copy(rows, acc.at[rowi], add=True)
      return carry
    lax.fori_loop(0, _NCHUNK, chunk_a, 0)

    plsc.subcore_barrier()
    drain(sum_h)
    plsc.subcore_barrier()

    zero_acc()
    plsc.subcore_barrier()

    def chunk_b(ci, carry):
      b = pl.multiple_of(ebase + ci * _K, 8)
      pltpu.sync_copy(row_h.at[pl.ds(b, _K)], rowi)
      pltpu.sync_copy(ob, acc.at[rowi], add=True)
      return carry
    lax.fori_loop(0, _NCHUNK, chunk_b, 0)

    plsc.subcore_barrier()
    drain(deg_h)

  return k(row, col, x, iota_arr)


_BR = 632


def _tc_finish(sums, degs, w, b2):
  def body(s_ref, d_ref, w_ref, b_ref, o_ref):
    s = s_ref[0] + s_ref[1]
    dg = d_ref[0, :, 0:1] + d_ref[1, :, 0:1]
    sn = s / dg
    o_ref[...] = lax.dot_general(
        sn, w_ref[...],
        dimension_numbers=(((1,), (1,)), ((), ())),
        preferred_element_type=jnp.float32) + b_ref[...]

  return pl.pallas_call(
      body,
      grid=(_NP // _BR,),
      in_specs=[
          pl.BlockSpec((2, _BR, _D), lambda i: (0, i, 0)),
          pl.BlockSpec((2, _BR, _D), lambda i: (0, i, 0)),
          pl.BlockSpec((_D, _D), lambda i: (0, 0)),
          pl.BlockSpec((1, _D), lambda i: (0, 0)),
      ],
      out_specs=pl.BlockSpec((_BR, _D), lambda i: (i, 0)),
      out_shape=jax.ShapeDtypeStruct((_NP, _D), jnp.float32),
  )(sums, degs, w, b2)


def kernel(x, edge_index, W, b):
  iota_arr = jnp.arange(_NP, dtype=jnp.int32)
  sums, degs = _sc_scatter(edge_index[0], edge_index[1], x, iota_arr)
  return _tc_finish(sums, degs, W, b.reshape(1, _D))[:_N]


# speedup vs baseline: 4.8679x; 1.2290x over previous
import functools

import jax
import jax.numpy as jnp
from jax import lax
from jax.experimental import pallas as pl
from jax.experimental.pallas import tpu as pltpu
from jax.experimental.pallas import tpu_sc as plsc

_N = 10000
_E = 320000
_D = 128
_NC = 2
_NS = 16
_NW = _NC * _NS
_EPW = _E // _NW
_K = 128
_NCHUNK = _EPW // _K
_KT = _EPW - _NCHUNK * _K
_NP = 10112
_RPT = _NP // _NS

_RCHUNK = tuple(min(i * _K, _RPT - _K) for i in range(5))


def _sc_scatter(row, col, x, iota_arr):
  mesh = plsc.VectorSubcoreMesh(core_axis_name="c", subcore_axis_name="s")

  @functools.partial(
      pl.kernel,
      mesh=mesh,
      out_type=[
          jax.ShapeDtypeStruct((_NC, _NP, _D), jnp.float32),
          jax.ShapeDtypeStruct((_NC, _NP, _D), jnp.float32),
      ],
      scratch_types=[
          pltpu.VMEM((_K,), jnp.int32),
          pltpu.VMEM((_K,), jnp.int32),
          pltpu.VMEM((_K,), jnp.int32),
          pltpu.VMEM((_KT,), jnp.int32),
          pltpu.VMEM((_KT,), jnp.int32),
          pltpu.VMEM((_K, _D), jnp.float32),
          pltpu.VMEM((_K, _D), jnp.float32),
          pltpu.VMEM_SHARED((_NP, _D), jnp.float32),
          pltpu.SemaphoreType.DMA,
      ],
  )
  def k(row_h, col_h, x_h, iota_h, sum_h, deg_h,
        rowi, coli, iot, rowt, colt, rows, ob, acc, sem):
    cid = lax.axis_index("c")
    sid = lax.axis_index("s")
    wid = sid * _NC + cid

    zv = jnp.zeros((16,), jnp.float32)
    ov = jnp.ones((16,), jnp.float32)

    def fill_rows_zero():
      def body(r, carry):
        for cc in range(_D // 16):
          rows[r, pl.ds(cc * 16, 16)] = zv
        return carry
      lax.fori_loop(0, _K, body, 0)

    def initv(r, carry):
      for cc in range(_D // 16):
        ob[r, pl.ds(cc * 16, 16)] = ov
      return carry
    lax.fori_loop(0, _K, initv, 0)

    base_r = sid * _RPT
    ebase = wid * _EPW

    def zero_acc():
      fill_rows_zero()
      for off_s in _RCHUNK:
        off = pl.multiple_of(base_r + off_s, 8)
        pltpu.sync_copy(iota_h.at[pl.ds(off, _K)], iot)
        pltpu.sync_copy(rows, acc.at[iot])

    def drain(out_h):
      for off_s in _RCHUNK:
        off = pl.multiple_of(base_r + off_s, 8)
        pltpu.sync_copy(iota_h.at[pl.ds(off, _K)], iot)
        pltpu.async_copy(acc.at[iot], rows, sem).wait()
        pltpu.sync_copy(rows, out_h.at[cid, pl.ds(off, _K)])

    zero_acc()
    plsc.subcore_barrier()

    def chunk_a(ci, carry):
      b = pl.multiple_of(ebase + ci * _K, 8)
      pltpu.sync_copy(row_h.at[pl.ds(b, _K)], rowi)
      pltpu.sync_copy(col_h.at[pl.ds(b, _K)], coli)
      pltpu.async_copy(x_h.at[coli], rows, sem).wait()
      pltpu.sync_copy(rows, acc.at[rowi], add=True)
      return carry
    lax.fori_loop(0, _NCHUNK, chunk_a, 0)
    bt = pl.multiple_of(ebase + _NCHUNK * _K, 8)
    pltpu.sync_copy(row_h.at[pl.ds(bt, _KT)], rowt)
    pltpu.sync_copy(col_h.at[pl.ds(bt, _KT)], colt)
    pltpu.async_copy(x_h.at[colt], rows.at[pl.ds(0, _KT)], sem).wait()
    pltpu.sync_copy(rows.at[pl.ds(0, _KT)], acc.at[rowt], add=True)

    plsc.subcore_barrier()
    drain(sum_h)
    plsc.subcore_barrier()

    zero_acc()
    plsc.subcore_barrier()

    def chunk_b(ci, carry):
      b = pl.multiple_of(ebase + ci * _K, 8)
      pltpu.sync_copy(row_h.at[pl.ds(b, _K)], rowi)
      pltpu.sync_copy(ob, acc.at[rowi], add=True)
      return carry
    lax.fori_loop(0, _NCHUNK, chunk_b, 0)
    bt2 = pl.multiple_of(ebase + _NCHUNK * _K, 8)
    pltpu.sync_copy(row_h.at[pl.ds(bt2, _KT)], rowt)
    pltpu.sync_copy(ob.at[pl.ds(0, _KT)], acc.at[rowt], add=True)

    plsc.subcore_barrier()
    drain(deg_h)

  return k(row, col, x, iota_arr)


_BR = 632


def _tc_finish(sums, degs, w, b2):
  def body(s_ref, d_ref, w_ref, b_ref, o_ref):
    s = s_ref[0] + s_ref[1]
    dg = d_ref[0, :, 0:1] + d_ref[1, :, 0:1]
    sn = s / dg
    o_ref[...] = lax.dot_general(
        sn, w_ref[...],
        dimension_numbers=(((1,), (1,)), ((), ())),
        preferred_element_type=jnp.float32) + b_ref[...]

  return pl.pallas_call(
      body,
      grid=(_NP // _BR,),
      in_specs=[
          pl.BlockSpec((2, _BR, _D), lambda i: (0, i, 0)),
          pl.BlockSpec((2, _BR, _D), lambda i: (0, i, 0)),
          pl.BlockSpec((_D, _D), lambda i: (0, 0)),
          pl.BlockSpec((1, _D), lambda i: (0, 0)),
      ],
      out_specs=pl.BlockSpec((_BR, _D), lambda i: (i, 0)),
      out_shape=jax.ShapeDtypeStruct((_NP, _D), jnp.float32),
  )(sums, degs, w, b2)


def kernel(x, edge_index, W, b):
  iota_arr = jnp.arange(_NP, dtype=jnp.int32)
  sums, degs = _sc_scatter(edge_index[0], edge_index[1], x, iota_arr)
  return _tc_finish(sums, degs, W, b.reshape(1, _D))[:_N]
